# Initial kernel scaffold; baseline (speedup 1.0000x reference)
import jax
import jax.numpy as jnp
from jax.experimental import pallas as pl


def kernel(data, params):
    raise NotImplementedError("write your pallas kernel here")


# speedup vs baseline: 2.0606x; 2.0606x over previous
import jax
import jax.numpy as jnp
from jax.experimental import pallas as pl
from jax.experimental.pallas import tpu as pltpu

_EMB = 512
_NE = 3
_DYN = [64, 32, 16, 8]
_TILE = 1024
_PREC = jax.lax.Precision.DEFAULT


def _dot(a, b, dims):
    return jax.lax.dot_general(a, b, dimension_numbers=(dims, ((), ())),
                               precision=_PREC,
                               preferred_element_type=jnp.float32)


def _layernorm(x, g, b, eps=1e-5):
    m = jnp.mean(x, axis=-1, keepdims=True)
    xc = x - m
    v = jnp.mean(xc * xc, axis=-1, keepdims=True)
    return xc / jnp.sqrt(v + eps) * g + b


def _leaky(x):
    return jnp.where(x >= 0, x, 0.01 * x)


def _ce_chain_body(cc_ref, cpW_ref, cpb_ref, *refs):
    red = refs[:4 * _NE]
    outs = refs[4 * _NE:]
    ce_outs, cn_outs = outs[:_NE + 1], outs[_NE + 1:]
    z = _dot(cc_ref[...], cpW_ref[...], ((1,), (1,))) + cpb_ref[...]
    ce = _leaky(z)
    ones = jnp.ones((1, _EMB), jnp.float32)
    for i in range(_NE):
        ce_outs[i][...] = ce
        cn_outs[i][...] = _dot(ones, ce * ce, ((1,), (1,)))
        w1, b1, w2, b2 = red[4 * i:4 * i + 4]
        u = jnp.maximum(_dot(w1[...], ce, ((1,), (0,))) + b1[...], 0.0)
        ce = _dot(w2[...], u, ((1,), (0,))) + b2[...]
    ce_outs[_NE][...] = ce


def _ce_chain(p):
    f32 = jnp.float32
    ins = [p['cluster_centers'], p['cp_W'], p['cp_b'].reshape(1, -1)]
    for i in range(_NE):
        ins += [p[f'red{i}_W1'], p[f'red{i}_b1'].reshape(-1, 1),
                p[f'red{i}_W2'], p[f'red{i}_b2'].reshape(-1, 1)]
    out_shape = ([jax.ShapeDtypeStruct((_DYN[i], _EMB), f32) for i in range(_NE + 1)]
                 + [jax.ShapeDtypeStruct((1, _DYN[i]), f32) for i in range(_NE)])
    return pl.pallas_call(_ce_chain_body, out_shape=out_shape)(*ins)


def _main_body(nt, data_ref, *refs):
    it = iter(refs)
    ppW, ppb, sscale = next(it), next(it), next(it)
    lnp = [(next(it), next(it), next(it), next(it), next(it), next(it))
           for _ in range(_NE)]
    (fpW, fpb, ang, anb, ftW, ftb, agW, agb, asW, asb,
     finW, finb, clsW, clsb) = (next(it) for _ in range(14))
    ces = [next(it) for _ in range(_NE + 1)]
    cns = [next(it) for _ in range(_NE)]
    logits_o, probs_o, yhat_o = next(it), next(it), next(it)
    M_s, S_s, V_s = next(it), next(it), next(it)
    k = pl.program_id(0)

    @pl.when(k == 0)
    def _init():
        M_s[0, 0] = -jnp.inf
        S_s[0, 0] = 0.0
        V_s[...] = jnp.zeros((1, _EMB), jnp.float32)

    asb_s = asb[0, 0]

    def tail(rows):
        z = jnp.maximum(_dot(rows, fpW[...], ((1,), (1,))) + fpb[...], 0.0)
        agg = _layernorm(z, ang[...], anb[...])
        tr = jnp.tanh(_dot(agg, ftW[...], ((1,), (1,))) + ftb[...])
        gt = jax.nn.sigmoid(_dot(agg, agW[...], ((1,), (1,))) + agb[...])
        s = jnp.sum((tr * gt) * asW[...], axis=1, keepdims=True) + asb_s
        return agg, s

    def accumulate(agg, s):
        m_tile = jnp.max(s)
        m_old = M_s[0, 0]
        m_new = jnp.maximum(m_old, m_tile)
        c = jnp.exp(m_old - m_new)
        w = jnp.exp(s - m_new)
        S_s[0, 0] = S_s[0, 0] * c + jnp.sum(w)
        V_s[...] = V_s[...] * c + jnp.sum(w * agg, axis=0, keepdims=True)
        M_s[0, 0] = m_new

    x = data_ref[...]
    pe = _leaky(_dot(x, ppW[...], ((1,), (1,))) + ppb[...])
    scale = sscale[0, 0]
    for i in range(_NE):
        lng, lnb, f1W, f1b, f2W, f2b = lnp[i]
        ce = ces[i][...]
        pn = jnp.sum(pe * pe, axis=1, keepdims=True)
        G = _dot(pe, ce, ((1,), (1,)))
        d2 = pn + cns[i][...] - 2.0 * G
        sim = -jnp.sqrt(jnp.maximum(d2, 1e-12)) / scale
        mx = jnp.max(sim, axis=1, keepdims=True)
        e = jnp.exp(sim - mx)
        a = e / jnp.sum(e, axis=1, keepdims=True)
        pe = pe + _dot(a, ce, ((1,), (0,)))
        pe = _layernorm(pe, lng[...], lnb[...])
        h = jnp.maximum(_dot(pe, f1W[...], ((1,), (1,))) + f1b[...], 0.0)
        pe = pe + _dot(h, f2W[...], ((1,), (1,))) + f2b[...]

    agg, s = tail(pe)
    accumulate(agg, s)

    @pl.when(k == nt - 1)
    def _finish():
        agg_c, s_c = tail(ces[_NE][...])
        accumulate(agg_c, s_c)
        slide = V_s[...] / S_s[0, 0]
        fin = jnp.maximum(_dot(slide, finW[...], ((1,), (1,))) + finb[...], 0.0)
        lg = _dot(fin, clsW[...], ((1,), (1,))) + clsb[...]
        logits_o[...] = lg
        mxl = jnp.max(lg)
        el = jnp.exp(lg - mxl)
        probs_o[...] = el / jnp.sum(el)
        iota = jax.lax.broadcasted_iota(jnp.int32, lg.shape, 1)
        idx = jnp.min(jnp.where(lg == mxl, iota, lg.shape[1]))
        yhat_o[...] = jnp.full((1, 1), idx, jnp.int32)


def kernel(data, params):
    p = params
    f32 = jnp.float32
    n = data.shape[0]
    nt = n // _TILE
    ce_cn = _ce_chain(p)

    r1 = lambda v: v.reshape(1, -1)
    ops = [p['pp_W'], r1(p['pp_b']), r1(p['sim_scale'])]
    for i in range(_NE):
        ops += [r1(p[f'ln{i}_g']), r1(p[f'ln{i}_b']),
                p[f'enh{i}_fc1_W'], r1(p[f'enh{i}_fc1_b']),
                p[f'enh{i}_fc2_W'], r1(p[f'enh{i}_fc2_b'])]
    ops += [p['fp_W'], r1(p['fp_b']), r1(p['an_g']), r1(p['an_b']),
            p['ft_W'], r1(p['ft_b']), p['ag_W'], r1(p['ag_b']),
            p['as_W'], r1(p['as_b']), p['final_W'], r1(p['final_b']),
            p['cls_W'], r1(p['cls_b'])]
    ops += list(ce_cn)

    const = lambda a: pl.BlockSpec(a.shape, lambda k: (0,) * a.ndim)
    smem = pl.BlockSpec(memory_space=pltpu.SMEM)
    in_specs = [pl.BlockSpec((_TILE, data.shape[1]), lambda k: (k, 0))]
    for j, a in enumerate(ops):
        in_specs.append(smem if a.shape == (1, 1) else const(a))
    out_shape = [jax.ShapeDtypeStruct((1, 4), f32),
                 jax.ShapeDtypeStruct((1, 4), f32),
                 jax.ShapeDtypeStruct((1, 1), jnp.int32)]
    out_specs = [pl.BlockSpec(s.shape, lambda k: (0, 0)) for s in out_shape]
    scratch = [pltpu.SMEM((1, 1), f32), pltpu.SMEM((1, 1), f32),
               pltpu.VMEM((1, _EMB), f32)]

    import functools
    logits, probs, yhat = pl.pallas_call(
        functools.partial(_main_body, nt),
        grid=(nt,),
        in_specs=in_specs,
        out_specs=out_specs,
        out_shape=out_shape,
        scratch_shapes=scratch,
    )(data, *ops)
    return logits, probs, yhat.reshape((1,))
